# Initial kernel scaffold; baseline (speedup 1.0000x reference)
import jax
import jax.numpy as jnp
from jax.experimental import pallas as pl


def kernel(n_feat, edge_index, Ws0, Wn0, b0, Ws1, Wn1, b1, Ws2, Wn2, b2):
    raise NotImplementedError("write your pallas kernel here")


# speedup vs baseline: 3.5874x; 3.5874x over previous
import jax
import jax.numpy as jnp
from jax import lax
from jax.experimental import pallas as pl
from jax.experimental.pallas import tpu as pltpu
from jax.experimental.pallas import tpu_sc as plsc

N_NODES = 10000
N_EDGES = 320000
D = 128

NC = 2
NS = 16
NW = NC * NS
E_W = N_EDGES // NW
CH = 80
N_CH = E_W // CH
N_PAD = 10240
ROWS_T = N_PAD // NS

def _make_sc_agg():
    mesh = plsc.VectorSubcoreMesh(
        core_axis_name="c", subcore_axis_name="s", num_cores=NC, num_subcores=NS
    )
    out_type = jax.ShapeDtypeStruct((NC, N_PAD, D), jnp.float32)
    scratch = [
        pltpu.VMEM((CH,), jnp.int32),
        pltpu.VMEM((CH,), jnp.int32),
        pltpu.VMEM((CH, D), jnp.float32),
        pltpu.VMEM_SHARED((N_PAD, D), jnp.float32),
        pltpu.SemaphoreType.DMA,
    ]

    def body(h_hbm, src_hbm, dst_hbm, znd_hbm, out_hbm,
             src_v, dst_v, rows_v, acc_s, sem):
        c = lax.axis_index("c")
        s = lax.axis_index("s")
        wid = s * NC + c
        r0 = s * ROWS_T

        for k in range(ROWS_T // CH):
            rr = r0 + k * CH
            pltpu.sync_copy(znd_hbm.at[pl.ds(rr, CH)], rows_v)
            pltpu.sync_copy(rows_v, acc_s.at[pl.ds(rr, CH)])
        plsc.subcore_barrier()

        base = wid * E_W

        def step(i, carry):
            off = base + i * CH
            pltpu.sync_copy(src_hbm.at[pl.ds(off, CH)], src_v)
            pltpu.sync_copy(dst_hbm.at[pl.ds(off, CH)], dst_v)
            pltpu.async_copy(h_hbm.at[src_v], rows_v, sem).wait()
            pltpu.sync_copy(rows_v, acc_s.at[dst_v], add=True)
            return carry

        lax.fori_loop(0, N_CH, step, 0)
        plsc.subcore_barrier()

        for k in range(ROWS_T // CH):
            rr = r0 + k * CH
            pltpu.sync_copy(acc_s.at[pl.ds(rr, CH)], rows_v)
            pltpu.sync_copy(rows_v, out_hbm.at[c, pl.ds(rr, CH)])

    return pl.kernel(body, out_type, mesh=mesh, scratch_types=scratch)


_sc_agg = _make_sc_agg()

_BR = 1000


def _tc_layer_body(h_ref, aA_ref, aB_ref, dA_ref, dB_ref, ws_ref, wn_ref,
                   b_ref, o_ref):
    deg = jnp.maximum(dA_ref[:, 0:1] + dB_ref[:, 0:1], 1.0)
    hn = (aA_ref[...] + aB_ref[...]) / deg
    acc = jnp.dot(h_ref[...], ws_ref[...], preferred_element_type=jnp.float32)
    acc = acc + jnp.dot(hn, wn_ref[...], preferred_element_type=jnp.float32)
    acc = acc + b_ref[...]
    o_ref[...] = jnp.maximum(acc, 0.01 * acc)


_tc_layer = pl.pallas_call(
    _tc_layer_body,
    grid=(N_NODES // _BR,),
    in_specs=[
        pl.BlockSpec((_BR, D), lambda i: (i, 0)),
        pl.BlockSpec((_BR, D), lambda i: (i, 0)),
        pl.BlockSpec((_BR, D), lambda i: (i, 0)),
        pl.BlockSpec((_BR, D), lambda i: (i, 0)),
        pl.BlockSpec((_BR, D), lambda i: (i, 0)),
        pl.BlockSpec((D, D), lambda i: (0, 0)),
        pl.BlockSpec((D, D), lambda i: (0, 0)),
        pl.BlockSpec((1, D), lambda i: (0, 0)),
    ],
    out_specs=pl.BlockSpec((_BR, D), lambda i: (i, 0)),
    out_shape=jax.ShapeDtypeStruct((N_NODES, D), jnp.float32),
)


def kernel(n_feat, edge_index, Ws0, Wn0, b0, Ws1, Wn1, b1, Ws2, Wn2, b2):
    src = edge_index[0].astype(jnp.int32)
    dst = edge_index[1].astype(jnp.int32)
    znd = jnp.zeros((N_PAD, D), jnp.float32)
    ones_feat = jnp.ones((N_NODES, D), jnp.float32)

    dgp = _sc_agg(ones_feat, src, dst, znd)
    dA, dB = dgp[0], dgp[1]

    h = n_feat
    agg = _sc_agg(h, src, dst, znd)
    h = _tc_layer(h, agg[0], agg[1], dA, dB, Ws0, Wn0, b0.reshape(1, D))
    agg = _sc_agg(h, src, dst, znd)
    h = _tc_layer(h, agg[0], agg[1], dA, dB, Ws1, Wn1, b1.reshape(1, D))
    agg = _sc_agg(h, src, dst, znd)
    h = _tc_layer(h, agg[0], agg[1], dA, dB, Ws2, Wn2, b2.reshape(1, D))
    return h
